# baseline (speedup 1.0000x reference)
import jax
import jax.numpy as jnp
from jax.experimental import pallas as pl
from jax.experimental.pallas import tpu as pltpu

_T, _A, _E, _D, _F = 8, 2, 8, 1024, 2816
_FH = _F // 2
_DH = _D // 2
_NC = 6
_LOOK = 7
_TOTAL = _E * _NC
_DIMS = (((1,), (1,)), ((), ()))


def _ffn_kernel(idx_ref, x_ref, w1_ref, w2_ref, w3_ref, out_ref,
                b13_ref, b2_ref, h_ref, sched_ref, sem13, sem2):
    present = []
    for e in range(_E):
        hit = idx_ref[0] == e
        for p in range(1, _T * _A):
            hit = hit | (idx_ref[p] == e)
        present.append(hit.astype(jnp.int32))
    for k in range(_E):
        sched_ref[k] = k
    sched_ref[_E] = _E

    n_used = sched_ref[_E]

    def copy_op(g):
        e, j = g // _NC, g % _NC
        expert = sched_ref[e]
        if j < 4:
            half, which = j // 2, j % 2
            src_ref = w1_ref if which == 0 else w3_ref
            return pltpu.make_async_copy(
                src_ref.at[expert, pl.ds(half * _FH, _FH), :],
                b13_ref.at[(4 * e + j) % 6], sem13.at[(4 * e + j) % 6])
        half = j - 4
        return pltpu.make_async_copy(
            w2_ref.at[expert, pl.ds(half * _DH, _DH), :],
            b2_ref.at[(2 * e + half) % 3], sem2.at[(2 * e + half) % 3])

    def issue(g):
        e = g // _NC

        @pl.when(e < n_used)
        def _():
            copy_op(g).start()

    def consume(g):
        e, j = g // _NC, g % _NC
        expert = sched_ref[e]

        @pl.when(e < n_used)
        def _():
            copy_op(g).wait()
            if j == 1 or j == 3:
                half = j // 2
                xb = x_ref[...]
                x1 = jax.lax.dot_general(
                    xb, b13_ref[(4 * e + 2 * half) % 6], _DIMS,
                    preferred_element_type=jnp.float32)
                x3 = jax.lax.dot_general(
                    xb, b13_ref[(4 * e + 2 * half + 1) % 6], _DIMS,
                    preferred_element_type=jnp.float32)
                h_ref[:, half * _FH:(half + 1) * _FH] = (
                    (x1 * jax.nn.sigmoid(x1)) * x3)
            elif j >= 4:
                half = j - 4
                res = jax.lax.dot_general(
                    h_ref[...], b2_ref[(2 * e + half) % 3], _DIMS,
                    preferred_element_type=jnp.float32)
                for p in range(_T * _A):
                    @pl.when(idx_ref[p] == expert)
                    def _write():
                        out_ref[p, half * _DH:(half + 1) * _DH] = (
                            res[p // _A, :])

    for g in range(_LOOK):
        issue(g)
    for g in range(_TOTAL):
        consume(g)
        if g + _LOOK < _TOTAL:
            issue(g + _LOOK)


def kernel(x, expert_indices, w1, w2, w3):
    idx = expert_indices.reshape(-1).astype(jnp.int32)

    hbm = pl.BlockSpec(memory_space=pltpu.MemorySpace.HBM)
    grid_spec = pltpu.PrefetchScalarGridSpec(
        num_scalar_prefetch=1,
        grid=(1,),
        in_specs=[
            pl.BlockSpec((_T, _D), lambda i, s1: (0, 0)),
            hbm, hbm, hbm,
        ],
        out_specs=pl.BlockSpec((_T * _A, _D), lambda i, s1: (0, 0)),
        scratch_shapes=[
            pltpu.VMEM((6, _FH, _D), jnp.float32),
            pltpu.VMEM((3, _DH, _F), jnp.float32),
            pltpu.VMEM((_T, _F), jnp.float32),
            pltpu.SMEM((_E + 1,), jnp.int32),
            pltpu.SemaphoreType.DMA((6,)),
            pltpu.SemaphoreType.DMA((3,)),
        ],
    )
    out = pl.pallas_call(
        _ffn_kernel,
        grid_spec=grid_spec,
        out_shape=jax.ShapeDtypeStruct((_T * _A, _D), jnp.float32),
    )(idx, x, w1, w2, w3)
    return out.reshape(_T, _A, _D)


# speedup vs baseline: 1.5174x; 1.5174x over previous
import jax
import jax.numpy as jnp
from jax.experimental import pallas as pl
from jax.experimental.pallas import tpu as pltpu

_T, _A, _E, _D, _F = 8, 2, 8, 1024, 2816
_FH = _F // 2
_DH = _D // 2
_NC = 6
_LOOK = 7
_TOTAL = _E * _NC
_DIMS = (((1,), (1,)), ((), ()))


def _ffn_kernel(idx_ref, x_ref, w1_ref, w2_ref, w3_ref, out_ref,
                b13_ref, b2_ref, h_ref, sched_ref, sem13, sem2):
    present = []
    for e in range(_E):
        hit = idx_ref[0] == e
        for p in range(1, _T * _A):
            hit = hit | (idx_ref[p] == e)
        present.append(hit.astype(jnp.int32))
    cnt = jnp.int32(0)
    for e in range(_E):
        pos = cnt
        for k in range(_E):
            @pl.when((present[e] == 1) & (pos == k))
            def _set():
                sched_ref[k] = e
        cnt = cnt + present[e]
    sched_ref[_E] = cnt

    n_used = sched_ref[_E]

    def copy_op(g):
        e, j = g // _NC, g % _NC
        expert = sched_ref[e]
        if j < 4:
            half, which = j // 2, j % 2
            src_ref = w1_ref if which == 0 else w3_ref
            return pltpu.make_async_copy(
                src_ref.at[expert, pl.ds(half * _FH, _FH), :],
                b13_ref.at[(4 * e + j) % 6], sem13.at[(4 * e + j) % 6])
        half = j - 4
        return pltpu.make_async_copy(
            w2_ref.at[expert, pl.ds(half * _DH, _DH), :],
            b2_ref.at[(2 * e + half) % 3], sem2.at[(2 * e + half) % 3])

    def issue(g):
        e = g // _NC

        @pl.when(e < n_used)
        def _():
            copy_op(g).start()

    def consume(g):
        e, j = g // _NC, g % _NC
        expert = sched_ref[e]

        @pl.when(e < n_used)
        def _():
            copy_op(g).wait()
            if j == 1 or j == 3:
                half = j // 2
                xb = x_ref[...]
                x1 = jax.lax.dot_general(
                    xb, b13_ref[(4 * e + 2 * half) % 6], _DIMS,
                    preferred_element_type=jnp.float32)
                x3 = jax.lax.dot_general(
                    xb, b13_ref[(4 * e + 2 * half + 1) % 6], _DIMS,
                    preferred_element_type=jnp.float32)
                h_ref[:, half * _FH:(half + 1) * _FH] = (
                    (x1 * jax.nn.sigmoid(x1)) * x3)
            elif j >= 4:
                half = j - 4
                res = jax.lax.dot_general(
                    h_ref[...], b2_ref[(2 * e + half) % 3], _DIMS,
                    preferred_element_type=jnp.float32)
                for p in range(_T * _A):
                    @pl.when(idx_ref[p] == expert)
                    def _write():
                        out_ref[p, half * _DH:(half + 1) * _DH] = (
                            res[p // _A, :])

    for g in range(_LOOK):
        issue(g)
    for g in range(_TOTAL):
        consume(g)
        if g + _LOOK < _TOTAL:
            issue(g + _LOOK)


def kernel(x, expert_indices, w1, w2, w3):
    idx = expert_indices.reshape(-1).astype(jnp.int32)

    hbm = pl.BlockSpec(memory_space=pltpu.MemorySpace.HBM)
    grid_spec = pltpu.PrefetchScalarGridSpec(
        num_scalar_prefetch=1,
        grid=(1,),
        in_specs=[
            pl.BlockSpec((_T, _D), lambda i, s1: (0, 0)),
            hbm, hbm, hbm,
        ],
        out_specs=pl.BlockSpec((_T * _A, _D), lambda i, s1: (0, 0)),
        scratch_shapes=[
            pltpu.VMEM((6, _FH, _D), jnp.float32),
            pltpu.VMEM((3, _DH, _F), jnp.float32),
            pltpu.VMEM((_T, _F), jnp.float32),
            pltpu.SMEM((_E + 1,), jnp.int32),
            pltpu.SemaphoreType.DMA((6,)),
            pltpu.SemaphoreType.DMA((3,)),
        ],
    )
    out = pl.pallas_call(
        _ffn_kernel,
        grid_spec=grid_spec,
        out_shape=jax.ShapeDtypeStruct((_T * _A, _D), jnp.float32),
    )(idx, x, w1, w2, w3)
    return out.reshape(_T, _A, _D)
